# baseline (speedup 1.0000x reference)
import dataclasses
import functools

import jax
import jax.numpy as jnp
from jax import lax
from jax.experimental import pallas as pl
from jax.experimental.pallas import tpu as pltpu
from jax.experimental.pallas import tpu_sc as plsc

_N = 10000
_NP = 10240
_E = 320000
_DIN = 128
_HID = 256
_NCLS = 40
_WPAD = 48

_NC = 2
_NS = 16
_NW = _NC * _NS
_B = 128
_NCHT = 2560
_EP = _NCHT * _B
_NCH = _NCHT // _NW
_RPS = _NP // _NS

_mesh = plsc.VectorSubcoreMesh(core_axis_name="c", subcore_axis_name="s")

_sc_params = pltpu.CompilerParams(use_tc_tiling_on_sc=False)
if "needs_layout_passes" in pltpu.CompilerParams.__dataclass_fields__:
    _sc_params = dataclasses.replace(_sc_params, needs_layout_passes=False)


def _make_agg(width, npass):
    nsub = width // 16

    @functools.partial(
        pl.kernel,
        out_type=jax.ShapeDtypeStruct((_NC, npass, _NP, width), jnp.float32),
        mesh=_mesh,
        compiler_params=_sc_params,
        scratch_types=[
            pltpu.VMEM((_NCH, _B), jnp.int32),
            pltpu.VMEM((_NCH, _B), jnp.int32),
            pltpu.VMEM((_NCH, _B), jnp.float32),
            pltpu.VMEM((4, _B, width), jnp.float32),
            pltpu.VMEM_SHARED((_NP, width), jnp.float32),
        ] + [pltpu.SemaphoreType.DMA] * 8,
    )
    def agg(*refs):
        y_hbms = refs[:npass]
        src_hbm, dst_hbm, ew_hbm, out_hbm = refs[npass:npass + 4]
        src_v, dst_v, ew_v, rows_v, acc_s = refs[npass + 4:npass + 9]
        gsem = refs[npass + 9:npass + 13]
        ssem = refs[npass + 13:npass + 17]
        cid = lax.axis_index("c")
        sid = lax.axis_index("s")
        wid = cid * _NS + sid

        pltpu.sync_copy(src_hbm.at[pl.ds(wid * _NCH, _NCH)], src_v)
        pltpu.sync_copy(dst_hbm.at[pl.ds(wid * _NCH, _NCH)], dst_v)
        pltpu.sync_copy(ew_hbm.at[pl.ds(wid * _NCH, _NCH)], ew_v)

        zero = jnp.zeros((16,), jnp.float32)

        for h in range(npass):
            @pl.loop(0, _B)
            def _(r):
                for k in range(nsub):
                    rows_v.at[0][r, pl.ds(k * 16, 16)] = zero

            for z in range(_RPS // _B):
                pltpu.sync_copy(rows_v.at[0],
                                acc_s.at[pl.ds(sid * _RPS + z * _B, _B)])
            plsc.subcore_barrier()

            y_hbm = y_hbms[h]

            def start_gather(c, j):
                pltpu.async_copy(y_hbm.at[src_v.at[c]], rows_v.at[j], gsem[j])

            def wait_gather(j):
                pltpu.make_async_copy(
                    y_hbm.at[src_v.at[0]], rows_v.at[j], gsem[j]).wait()

            def start_scatter(c, j):
                pltpu.async_copy(rows_v.at[j], acc_s.at[dst_v.at[c]],
                                 ssem[j], add=True)

            def wait_scatter(j):
                pltpu.make_async_copy(
                    rows_v.at[j], acc_s.at[dst_v.at[0]], ssem[j]).wait()

            def compute(c, j):
                @plsc.parallel_loop(0, _B, unroll=4)
                def _(e):
                    w = plsc.load_gather(
                        ew_v, [lax.broadcast_in_dim(c, (16,), ()),
                               lax.broadcast_in_dim(e, (16,), ())])
                    for k in range(nsub):
                        sl = (e, pl.ds(k * 16, 16))
                        rows_v.at[j][sl] = rows_v.at[j][sl] * w

            start_gather(0, 0)
            start_gather(1, 1)
            wait_gather(0)
            compute(0, 0)
            start_scatter(0, 0)
            start_gather(2, 2)
            wait_gather(1)
            compute(1, 1)
            start_scatter(1, 1)
            start_gather(3, 3)

            @pl.loop(0, 19)
            def _(m):
                for j in range(4):
                    c = 4 * m + 2 + j
                    b = (j + 2) % 4
                    bn = j
                    wait_gather(b)
                    compute(c, b)
                    start_scatter(c, b)
                    wait_scatter(bn)
                    start_gather(c + 2, bn)

            wait_gather(2)
            compute(_NCH - 2, 2)
            start_scatter(_NCH - 2, 2)
            wait_scatter(0)
            wait_gather(3)
            compute(_NCH - 1, 3)
            start_scatter(_NCH - 1, 3)
            wait_scatter(1)
            wait_scatter(2)
            wait_scatter(3)

            plsc.subcore_barrier()
            pltpu.sync_copy(acc_s.at[pl.ds(sid * _RPS, _RPS)],
                            out_hbm.at[cid, h, pl.ds(sid * _RPS, _RPS)])

    return agg


_HW = _DIN // 2
_agg_l1 = _make_agg(_HW, 2)
_agg_l2 = _make_agg(_WPAD, 1)


@functools.partial(
    pl.kernel,
    out_type=jax.ShapeDtypeStruct((_NC, _NP, 16), jnp.float32),
    mesh=_mesh,
    compiler_params=_sc_params,
    scratch_types=[
        pltpu.VMEM((_NCH, _B), jnp.int32),
        pltpu.VMEM((_NCH, _B), jnp.float32),
        pltpu.VMEM((2, _B, 16), jnp.float32),
        pltpu.VMEM_SHARED((_NP, 16), jnp.float32),
    ] + [pltpu.SemaphoreType.DMA] * 2,
)
def _deg_kernel(dst_hbm, ew_hbm, out_hbm, dst_v, ew_v, rows_v, acc_s,
                ssem0, ssem1):
    ssem = (ssem0, ssem1)
    cid = lax.axis_index("c")
    sid = lax.axis_index("s")
    wid = cid * _NS + sid

    pltpu.sync_copy(dst_hbm.at[pl.ds(wid * _NCH, _NCH)], dst_v)
    pltpu.sync_copy(ew_hbm.at[pl.ds(wid * _NCH, _NCH)], ew_v)

    zero = jnp.zeros((16,), jnp.float32)

    @pl.loop(0, _B)
    def _(r):
        rows_v.at[0][r, :] = zero

    for z in range(_RPS // _B):
        pltpu.sync_copy(rows_v.at[0], acc_s.at[pl.ds(sid * _RPS + z * _B, _B)])
    plsc.subcore_barrier()

    def build(c, j):
        @plsc.parallel_loop(0, _B, unroll=4)
        def _(e):
            w = plsc.load_gather(
                ew_v, [lax.broadcast_in_dim(c, (16,), ()),
                       lax.broadcast_in_dim(e, (16,), ())])
            rows_v.at[j][e, :] = w

    def start_scatter(c, j):
        pltpu.async_copy(rows_v.at[j], acc_s.at[dst_v.at[c]],
                         ssem[j], add=True)

    def wait_scatter(j):
        pltpu.make_async_copy(rows_v.at[j], acc_s.at[dst_v.at[0]],
                              ssem[j]).wait()

    build(0, 0)
    start_scatter(0, 0)
    build(1, 1)
    start_scatter(1, 1)

    @pl.loop(0, (_NCH - 2) // 2)
    def _(m):
        for j in range(2):
            c = 2 * m + 2 + j
            wait_scatter(j)
            build(c, j)
            start_scatter(c, j)

    wait_scatter(0)
    wait_scatter(1)

    plsc.subcore_barrier()
    pltpu.sync_copy(acc_s.at[pl.ds(sid * _RPS, _RPS)],
                    out_hbm.at[cid, pl.ds(sid * _RPS, _RPS)])


_R = 1024


def _tc0_body(degp_ref, x_ref, dinv_ref, y0_ref, y1_ref):
    deg = degp_ref[0, :, 0] + degp_ref[1, :, 0] + 1.0
    dinv = lax.rsqrt(deg)
    dinv_ref[...] = dinv[:, None]
    y = x_ref[...] * dinv[:, None]
    y0_ref[...] = y[:, :_HW]
    y1_ref[...] = y[:, _HW:]


_tc0 = pl.pallas_call(
    _tc0_body,
    grid=(_NP // _R,),
    in_specs=[
        pl.BlockSpec((2, _R, 16), lambda i: (0, i, 0)),
        pl.BlockSpec((_R, _DIN), lambda i: (i, 0)),
    ],
    out_specs=[
        pl.BlockSpec((_R, 1), lambda i: (i, 0)),
        pl.BlockSpec((_R, _HW), lambda i: (i, 0)),
        pl.BlockSpec((_R, _HW), lambda i: (i, 0)),
    ],
    out_shape=[
        jax.ShapeDtypeStruct((_NP, 1), jnp.float32),
        jax.ShapeDtypeStruct((_NP, _HW), jnp.float32),
        jax.ShapeDtypeStruct((_NP, _HW), jnp.float32),
    ],
)


def _tc1_body(a1p_ref, x_ref, dinv_ref, w1_ref, b1_ref, w2_ref, g_ref):
    dinv = dinv_ref[...]
    a1 = jnp.concatenate(
        [a1p_ref[0, 0] + a1p_ref[1, 0], a1p_ref[0, 1] + a1p_ref[1, 1]], axis=1)
    out1 = a1 * dinv + x_ref[...] * (dinv * dinv)
    h = jnp.dot(out1, w1_ref[...], precision=lax.Precision.HIGHEST)
    h = jnp.maximum(h + b1_ref[...], 0.0)
    p = jnp.dot(h, w2_ref[...], precision=lax.Precision.HIGHEST)
    g_ref[...] = p * dinv


_tc1 = pl.pallas_call(
    _tc1_body,
    grid=(_NP // _R,),
    in_specs=[
        pl.BlockSpec((2, 2, _R, _HW), lambda i: (0, 0, i, 0)),
        pl.BlockSpec((_R, _DIN), lambda i: (i, 0)),
        pl.BlockSpec((_R, 1), lambda i: (i, 0)),
        pl.BlockSpec((_DIN, _HID), lambda i: (0, 0)),
        pl.BlockSpec((_HID,), lambda i: (0,)),
        pl.BlockSpec((_HID, _WPAD), lambda i: (0, 0)),
    ],
    out_specs=pl.BlockSpec((_R, _WPAD), lambda i: (i, 0)),
    out_shape=jax.ShapeDtypeStruct((_NP, _WPAD), jnp.float32),
)


def _tc2_body(a2p_ref, g_ref, dinv_ref, b2_ref, o_ref):
    dinv = dinv_ref[...]
    s = (a2p_ref[0, 0] + a2p_ref[1, 0] + g_ref[...]) * dinv
    o_ref[...] = s[:, :_NCLS] + b2_ref[...]


_tc2 = pl.pallas_call(
    _tc2_body,
    grid=(_NP // _R,),
    in_specs=[
        pl.BlockSpec((2, 1, _R, _WPAD), lambda i: (0, 0, i, 0)),
        pl.BlockSpec((_R, _WPAD), lambda i: (i, 0)),
        pl.BlockSpec((_R, 1), lambda i: (i, 0)),
        pl.BlockSpec((_NCLS,), lambda i: (0,)),
    ],
    out_specs=pl.BlockSpec((_R, _NCLS), lambda i: (i, 0)),
    out_shape=jax.ShapeDtypeStruct((_NP, _NCLS), jnp.float32),
)


def kernel(x, edge_index, edge_attr, W1, b1, W2, b2):
    pad = _EP - _E
    src = jnp.pad(edge_index[0], (0, pad)).reshape(_NCHT, _B)
    dst = jnp.pad(edge_index[1], (0, pad)).reshape(_NCHT, _B)
    ew = jnp.pad(edge_attr, (0, pad)).reshape(_NCHT, _B)
    w2p = jnp.pad(W2, ((0, 0), (0, _WPAD - _NCLS)))
    xp = jnp.pad(x, ((0, _NP - _N), (0, 0)))

    degp = _deg_kernel(dst, ew)
    dinv, y0, y1 = _tc0(degp, xp)
    a1p = _agg_l1(y0, y1, src, dst, ew)
    g = _tc1(a1p, xp, dinv, W1, b1, w2p)
    a2p = _agg_l2(g, src, dst, ew)
    return _tc2(a2p, g, dinv, b2)[:_N]


# speedup vs baseline: 2.2366x; 2.2366x over previous
import dataclasses
import functools

import jax
import jax.numpy as jnp
from jax import lax
from jax.experimental import pallas as pl
from jax.experimental.pallas import tpu as pltpu
from jax.experimental.pallas import tpu_sc as plsc

_N = 10000
_NP = 10240
_E = 320000
_DIN = 128
_HID = 256
_NCLS = 40
_WPAD = 48

_NC = 2
_NS = 16
_NW = _NC * _NS
_B = 128
_NCHT = 2560
_EP = _NCHT * _B
_NCH = _NCHT // _NW
_RPS = _NP // _NS

_mesh = plsc.VectorSubcoreMesh(core_axis_name="c", subcore_axis_name="s")

_sc_params = pltpu.CompilerParams(use_tc_tiling_on_sc=False)
if "needs_layout_passes" in pltpu.CompilerParams.__dataclass_fields__:
    _sc_params = dataclasses.replace(_sc_params, needs_layout_passes=False)


def _make_agg(width, npass):
    nsub = width // 16

    @functools.partial(
        pl.kernel,
        out_type=jax.ShapeDtypeStruct((_NC, npass, _NP, width), jnp.float32),
        mesh=_mesh,
        compiler_params=_sc_params,
        scratch_types=[
            pltpu.VMEM((_NCH, _B), jnp.int32),
            pltpu.VMEM((_NCH, _B), jnp.int32),
            pltpu.VMEM((_NCH, _B), jnp.float32),
            pltpu.VMEM((4, _B, width), jnp.float32),
            pltpu.VMEM_SHARED((_NP, width), jnp.float32),
        ] + [pltpu.SemaphoreType.DMA] * 8,
    )
    def agg(*refs):
        y_hbms = refs[:npass]
        src_hbm, dst_hbm, ew_hbm, out_hbm = refs[npass:npass + 4]
        src_v, dst_v, ew_v, rows_v, acc_s = refs[npass + 4:npass + 9]
        gsem = refs[npass + 9:npass + 13]
        ssem = refs[npass + 13:npass + 17]
        cid = lax.axis_index("c")
        sid = lax.axis_index("s")
        wid = cid * _NS + sid

        pltpu.sync_copy(src_hbm.at[pl.ds(wid * _NCH, _NCH)], src_v)
        pltpu.sync_copy(dst_hbm.at[pl.ds(wid * _NCH, _NCH)], dst_v)
        pltpu.sync_copy(ew_hbm.at[pl.ds(wid * _NCH, _NCH)], ew_v)

        zero = jnp.zeros((16,), jnp.float32)

        for h in range(npass):
            @pl.loop(0, _B)
            def _(r):
                for k in range(nsub):
                    rows_v.at[0][r, pl.ds(k * 16, 16)] = zero

            for z in range(_RPS // _B):
                pltpu.sync_copy(rows_v.at[0],
                                acc_s.at[pl.ds(sid * _RPS + z * _B, _B)])
            plsc.subcore_barrier()

            y_hbm = y_hbms[h]

            def start_gather(c, j):
                pltpu.async_copy(y_hbm.at[src_v.at[c]], rows_v.at[j], gsem[j])

            def wait_gather(j):
                pltpu.make_async_copy(
                    y_hbm.at[src_v.at[0]], rows_v.at[j], gsem[j]).wait()

            def start_scatter(c, j):
                pltpu.async_copy(rows_v.at[j], acc_s.at[dst_v.at[c]],
                                 ssem[j], add=True)

            def wait_scatter(j):
                pltpu.make_async_copy(
                    rows_v.at[j], acc_s.at[dst_v.at[0]], ssem[j]).wait()

            def compute(c, j):
                cvec = lax.broadcast_in_dim(c, (16,), ())

                @plsc.parallel_loop(0, _B, unroll=8)
                def _(e):
                    w = plsc.load_gather(
                        ew_v, [cvec, lax.broadcast_in_dim(e, (16,), ())])
                    for k in range(nsub):
                        sl = (e, pl.ds(k * 16, 16))
                        rows_v.at[j][sl] = rows_v.at[j][sl] * w

            start_gather(0, 0)
            start_gather(1, 1)
            wait_gather(0)
            compute(0, 0)
            start_scatter(0, 0)
            start_gather(2, 2)
            wait_gather(1)
            compute(1, 1)
            start_scatter(1, 1)
            start_gather(3, 3)

            @pl.loop(0, 19)
            def _(m):
                for j in range(4):
                    c = 4 * m + 2 + j
                    b = (j + 2) % 4
                    bn = j
                    wait_gather(b)
                    compute(c, b)
                    start_scatter(c, b)
                    wait_scatter(bn)
                    start_gather(c + 2, bn)

            wait_gather(2)
            compute(_NCH - 2, 2)
            start_scatter(_NCH - 2, 2)
            wait_scatter(0)
            wait_gather(3)
            compute(_NCH - 1, 3)
            start_scatter(_NCH - 1, 3)
            wait_scatter(1)
            wait_scatter(2)
            wait_scatter(3)

            plsc.subcore_barrier()
            pltpu.sync_copy(acc_s.at[pl.ds(sid * _RPS, _RPS)],
                            out_hbm.at[cid, h, pl.ds(sid * _RPS, _RPS)])

    return agg


_HW = _DIN // 2
_agg_l1 = _make_agg(_HW, 2)
_agg_l2 = _make_agg(_WPAD, 1)


@functools.partial(
    pl.kernel,
    out_type=jax.ShapeDtypeStruct((_NC, _NP, 16), jnp.float32),
    mesh=_mesh,
    compiler_params=_sc_params,
    scratch_types=[
        pltpu.VMEM((_NCH, _B), jnp.int32),
        pltpu.VMEM((_NCH, _B), jnp.float32),
        pltpu.VMEM((2, _B, 16), jnp.float32),
        pltpu.VMEM_SHARED((_NP, 16), jnp.float32),
    ] + [pltpu.SemaphoreType.DMA] * 2,
)
def _deg_kernel(dst_hbm, ew_hbm, out_hbm, dst_v, ew_v, rows_v, acc_s,
                ssem0, ssem1):
    ssem = (ssem0, ssem1)
    cid = lax.axis_index("c")
    sid = lax.axis_index("s")
    wid = cid * _NS + sid

    pltpu.sync_copy(dst_hbm.at[pl.ds(wid * _NCH, _NCH)], dst_v)
    pltpu.sync_copy(ew_hbm.at[pl.ds(wid * _NCH, _NCH)], ew_v)

    zero = jnp.zeros((16,), jnp.float32)

    @pl.loop(0, _B)
    def _(r):
        rows_v.at[0][r, :] = zero

    for z in range(_RPS // _B):
        pltpu.sync_copy(rows_v.at[0], acc_s.at[pl.ds(sid * _RPS + z * _B, _B)])
    plsc.subcore_barrier()

    def build(c, j):
        cvec = lax.broadcast_in_dim(c, (16,), ())

        @plsc.parallel_loop(0, _B, unroll=8)
        def _(e):
            w = plsc.load_gather(
                ew_v, [cvec, lax.broadcast_in_dim(e, (16,), ())])
            rows_v.at[j][e, :] = w

    def start_scatter(c, j):
        pltpu.async_copy(rows_v.at[j], acc_s.at[dst_v.at[c]],
                         ssem[j], add=True)

    def wait_scatter(j):
        pltpu.make_async_copy(rows_v.at[j], acc_s.at[dst_v.at[0]],
                              ssem[j]).wait()

    build(0, 0)
    start_scatter(0, 0)
    build(1, 1)
    start_scatter(1, 1)

    @pl.loop(0, (_NCH - 2) // 2)
    def _(m):
        for j in range(2):
            c = 2 * m + 2 + j
            wait_scatter(j)
            build(c, j)
            start_scatter(c, j)

    wait_scatter(0)
    wait_scatter(1)

    plsc.subcore_barrier()
    pltpu.sync_copy(acc_s.at[pl.ds(sid * _RPS, _RPS)],
                    out_hbm.at[cid, pl.ds(sid * _RPS, _RPS)])


_R = 1024


def _tc0_body(degp_ref, x_ref, dinv_ref, y0_ref, y1_ref):
    deg = degp_ref[0, :, 0] + degp_ref[1, :, 0] + 1.0
    dinv = lax.rsqrt(deg)
    dinv_ref[...] = dinv[:, None]
    y = x_ref[...] * dinv[:, None]
    y0_ref[...] = y[:, :_HW]
    y1_ref[...] = y[:, _HW:]


_tc0 = pl.pallas_call(
    _tc0_body,
    grid=(_NP // _R,),
    in_specs=[
        pl.BlockSpec((2, _R, 16), lambda i: (0, i, 0)),
        pl.BlockSpec((_R, _DIN), lambda i: (i, 0)),
    ],
    out_specs=[
        pl.BlockSpec((_R, 1), lambda i: (i, 0)),
        pl.BlockSpec((_R, _HW), lambda i: (i, 0)),
        pl.BlockSpec((_R, _HW), lambda i: (i, 0)),
    ],
    out_shape=[
        jax.ShapeDtypeStruct((_NP, 1), jnp.float32),
        jax.ShapeDtypeStruct((_NP, _HW), jnp.float32),
        jax.ShapeDtypeStruct((_NP, _HW), jnp.float32),
    ],
)


def _tc1_body(a1p_ref, x_ref, dinv_ref, w1_ref, b1_ref, w2_ref, g_ref):
    dinv = dinv_ref[...]
    a1 = jnp.concatenate(
        [a1p_ref[0, 0] + a1p_ref[1, 0], a1p_ref[0, 1] + a1p_ref[1, 1]], axis=1)
    out1 = a1 * dinv + x_ref[...] * (dinv * dinv)
    h = jnp.dot(out1, w1_ref[...])
    h = jnp.maximum(h + b1_ref[...], 0.0)
    p = jnp.dot(h, w2_ref[...])
    g_ref[...] = p * dinv


_tc1 = pl.pallas_call(
    _tc1_body,
    grid=(_NP // _R,),
    in_specs=[
        pl.BlockSpec((2, 2, _R, _HW), lambda i: (0, 0, i, 0)),
        pl.BlockSpec((_R, _DIN), lambda i: (i, 0)),
        pl.BlockSpec((_R, 1), lambda i: (i, 0)),
        pl.BlockSpec((_DIN, _HID), lambda i: (0, 0)),
        pl.BlockSpec((_HID,), lambda i: (0,)),
        pl.BlockSpec((_HID, _WPAD), lambda i: (0, 0)),
    ],
    out_specs=pl.BlockSpec((_R, _WPAD), lambda i: (i, 0)),
    out_shape=jax.ShapeDtypeStruct((_NP, _WPAD), jnp.float32),
)


def _tc2_body(a2p_ref, g_ref, dinv_ref, b2_ref, o_ref):
    dinv = dinv_ref[...]
    s = (a2p_ref[0, 0] + a2p_ref[1, 0] + g_ref[...]) * dinv
    o_ref[...] = s[:, :_NCLS] + b2_ref[...]


_tc2 = pl.pallas_call(
    _tc2_body,
    grid=(_NP // _R,),
    in_specs=[
        pl.BlockSpec((2, 1, _R, _WPAD), lambda i: (0, 0, i, 0)),
        pl.BlockSpec((_R, _WPAD), lambda i: (i, 0)),
        pl.BlockSpec((_R, 1), lambda i: (i, 0)),
        pl.BlockSpec((_NCLS,), lambda i: (0,)),
    ],
    out_specs=pl.BlockSpec((_R, _NCLS), lambda i: (i, 0)),
    out_shape=jax.ShapeDtypeStruct((_NP, _NCLS), jnp.float32),
)


def kernel(x, edge_index, edge_attr, W1, b1, W2, b2):
    pad = _EP - _E
    fill = jnp.arange(pad, dtype=jnp.int32)
    src = jnp.concatenate([edge_index[0], fill]).reshape(_NCHT, _B)
    dst = jnp.concatenate([edge_index[1], fill]).reshape(_NCHT, _B)
    ew = jnp.pad(edge_attr, (0, pad)).reshape(_NCHT, _B)
    w2p = jnp.pad(W2, ((0, 0), (0, _WPAD - _NCLS)))
    xp = jnp.pad(x, ((0, _NP - _N), (0, 0)))

    degp = _deg_kernel(dst, ew)
    dinv, y0, y1 = _tc0(degp, xp)
    a1p = _agg_l1(y0, y1, src, dst, ew)
    g = _tc1(a1p, xp, dinv, W1, b1, w2p)
    a2p = _agg_l2(g, src, dst, ew)
    return _tc2(a2p, g, dinv, b2)[:_N]


# speedup vs baseline: 2.4305x; 1.0867x over previous
import dataclasses
import functools

import jax
import jax.numpy as jnp
from jax import lax
from jax.experimental import pallas as pl
from jax.experimental.pallas import tpu as pltpu
from jax.experimental.pallas import tpu_sc as plsc

_N = 10000
_NP = 10240
_E = 320000
_DIN = 128
_HID = 256
_NCLS = 40
_WPAD = 48

_NC = 2
_NS = 16
_NW = _NC * _NS
_B = 128
_NCHT = 2560
_EP = _NCHT * _B
_NCH = _NCHT // _NW
_RPS = _NP // _NS

_mesh = plsc.VectorSubcoreMesh(core_axis_name="c", subcore_axis_name="s")

_sc_params = pltpu.CompilerParams(use_tc_tiling_on_sc=False)
if "needs_layout_passes" in pltpu.CompilerParams.__dataclass_fields__:
    _sc_params = dataclasses.replace(_sc_params, needs_layout_passes=False)


def _make_agg(width, npass):
    nsub = width // 16

    @functools.partial(
        pl.kernel,
        out_type=jax.ShapeDtypeStruct((_NC, _NP, 128), jnp.float32),
        mesh=_mesh,
        compiler_params=_sc_params,
        scratch_types=[
            pltpu.VMEM((_NCH, _B), jnp.int32),
            pltpu.VMEM((_NCH, _B), jnp.int32),
            pltpu.VMEM((_NCH, _B), jnp.float32),
            pltpu.VMEM((4, _B, width), jnp.float32),
            pltpu.VMEM_SHARED((_NP, width), jnp.float32),
        ] + [pltpu.SemaphoreType.DMA] * 8,
    )
    def agg(*refs):
        y_hbms = refs[:npass]
        src_hbm, dst_hbm, ew_hbm, out_hbm = refs[npass:npass + 4]
        src_v, dst_v, ew_v, rows_v, acc_s = refs[npass + 4:npass + 9]
        gsem = refs[npass + 9:npass + 13]
        ssem = refs[npass + 13:npass + 17]
        cid = lax.axis_index("c")
        sid = lax.axis_index("s")
        wid = cid * _NS + sid

        pltpu.sync_copy(src_hbm.at[pl.ds(wid * _NCH, _NCH)], src_v)
        pltpu.sync_copy(dst_hbm.at[pl.ds(wid * _NCH, _NCH)], dst_v)
        pltpu.sync_copy(ew_hbm.at[pl.ds(wid * _NCH, _NCH)], ew_v)

        zero = jnp.zeros((16,), jnp.float32)

        for h in range(npass):
            @pl.loop(0, _B)
            def _(r):
                for k in range(nsub):
                    rows_v.at[0][r, pl.ds(k * 16, 16)] = zero

            for z in range(_RPS // _B):
                pltpu.sync_copy(rows_v.at[0],
                                acc_s.at[pl.ds(sid * _RPS + z * _B, _B)])
            plsc.subcore_barrier()

            y_hbm = y_hbms[h]

            def start_gather(c, j):
                pltpu.async_copy(y_hbm.at[src_v.at[c]], rows_v.at[j], gsem[j])

            def wait_gather(j):
                pltpu.make_async_copy(
                    y_hbm.at[src_v.at[0]], rows_v.at[j], gsem[j]).wait()

            def start_scatter(c, j):
                pltpu.async_copy(rows_v.at[j], acc_s.at[dst_v.at[c]],
                                 ssem[j], add=True)

            def wait_scatter(j):
                pltpu.make_async_copy(
                    rows_v.at[j], acc_s.at[dst_v.at[0]], ssem[j]).wait()

            def compute(c, j):
                cvec = lax.broadcast_in_dim(c, (16,), ())

                @plsc.parallel_loop(0, _B, unroll=8)
                def _(e):
                    w = plsc.load_gather(
                        ew_v, [cvec, lax.broadcast_in_dim(e, (16,), ())])
                    for k in range(nsub):
                        sl = (e, pl.ds(k * 16, 16))
                        rows_v.at[j][sl] = rows_v.at[j][sl] * w

            start_gather(0, 0)
            start_gather(1, 1)
            wait_gather(0)
            compute(0, 0)
            start_scatter(0, 0)
            start_gather(2, 2)
            wait_gather(1)
            compute(1, 1)
            start_scatter(1, 1)
            start_gather(3, 3)

            @pl.loop(0, 19)
            def _(m):
                for j in range(4):
                    c = 4 * m + 2 + j
                    b = (j + 2) % 4
                    bn = j
                    wait_gather(b)
                    compute(c, b)
                    start_scatter(c, b)
                    wait_scatter(bn)
                    start_gather(c + 2, bn)

            wait_gather(2)
            compute(_NCH - 2, 2)
            start_scatter(_NCH - 2, 2)
            wait_scatter(0)
            wait_gather(3)
            compute(_NCH - 1, 3)
            start_scatter(_NCH - 1, 3)
            wait_scatter(1)
            wait_scatter(2)
            wait_scatter(3)

            plsc.subcore_barrier()
            pltpu.sync_copy(
                acc_s.at[pl.ds(sid * _RPS, _RPS)],
                out_hbm.at[cid, pl.ds(sid * _RPS, _RPS),
                           pl.ds(h * width, width)])

    return agg


_HW = _DIN // 2
_agg_l1 = _make_agg(_HW, 2)
_agg_l2 = _make_agg(_WPAD, 1)


@functools.partial(
    pl.kernel,
    out_type=jax.ShapeDtypeStruct((_NC, _NP, 16), jnp.float32),
    mesh=_mesh,
    compiler_params=_sc_params,
    scratch_types=[
        pltpu.VMEM((_NCH, _B), jnp.int32),
        pltpu.VMEM((_NCH, _B), jnp.float32),
        pltpu.VMEM((2, _B, 16), jnp.float32),
        pltpu.VMEM_SHARED((_NP, 16), jnp.float32),
    ] + [pltpu.SemaphoreType.DMA] * 2,
)
def _deg_kernel(dst_hbm, ew_hbm, out_hbm, dst_v, ew_v, rows_v, acc_s,
                ssem0, ssem1):
    ssem = (ssem0, ssem1)
    cid = lax.axis_index("c")
    sid = lax.axis_index("s")
    wid = cid * _NS + sid

    pltpu.sync_copy(dst_hbm.at[pl.ds(wid * _NCH, _NCH)], dst_v)
    pltpu.sync_copy(ew_hbm.at[pl.ds(wid * _NCH, _NCH)], ew_v)

    zero = jnp.zeros((16,), jnp.float32)

    @pl.loop(0, _B)
    def _(r):
        rows_v.at[0][r, :] = zero

    for z in range(_RPS // _B):
        pltpu.sync_copy(rows_v.at[0], acc_s.at[pl.ds(sid * _RPS + z * _B, _B)])
    plsc.subcore_barrier()

    def build(c, j):
        cvec = lax.broadcast_in_dim(c, (16,), ())

        @plsc.parallel_loop(0, _B, unroll=8)
        def _(e):
            w = plsc.load_gather(
                ew_v, [cvec, lax.broadcast_in_dim(e, (16,), ())])
            rows_v.at[j][e, :] = w

    def start_scatter(c, j):
        pltpu.async_copy(rows_v.at[j], acc_s.at[dst_v.at[c]],
                         ssem[j], add=True)

    def wait_scatter(j):
        pltpu.make_async_copy(rows_v.at[j], acc_s.at[dst_v.at[0]],
                              ssem[j]).wait()

    build(0, 0)
    start_scatter(0, 0)
    build(1, 1)
    start_scatter(1, 1)

    @pl.loop(0, (_NCH - 2) // 2)
    def _(m):
        for j in range(2):
            c = 2 * m + 2 + j
            wait_scatter(j)
            build(c, j)
            start_scatter(c, j)

    wait_scatter(0)
    wait_scatter(1)

    plsc.subcore_barrier()
    pltpu.sync_copy(acc_s.at[pl.ds(sid * _RPS, _RPS)],
                    out_hbm.at[cid, pl.ds(sid * _RPS, _RPS)])


_R = 1024


def _tc0_body(degp_ref, x_ref, dinv_ref, y0_ref, y1_ref):
    deg = degp_ref[0, :, 0] + degp_ref[1, :, 0] + 1.0
    dinv = lax.rsqrt(deg)
    dinv_ref[...] = dinv[:, None]
    y = x_ref[...] * dinv[:, None]
    y0_ref[...] = y[:, :_HW]
    y1_ref[...] = y[:, _HW:]


_tc0 = pl.pallas_call(
    _tc0_body,
    grid=(_NP // _R,),
    in_specs=[
        pl.BlockSpec((2, _R, 16), lambda i: (0, i, 0)),
        pl.BlockSpec((_R, _DIN), lambda i: (i, 0)),
    ],
    out_specs=[
        pl.BlockSpec((_R, 1), lambda i: (i, 0)),
        pl.BlockSpec((_R, _HW), lambda i: (i, 0)),
        pl.BlockSpec((_R, _HW), lambda i: (i, 0)),
    ],
    out_shape=[
        jax.ShapeDtypeStruct((_NP, 1), jnp.float32),
        jax.ShapeDtypeStruct((_NP, _HW), jnp.float32),
        jax.ShapeDtypeStruct((_NP, _HW), jnp.float32),
    ],
)


def _tc1_body(a1p_ref, x_ref, dinv_ref, w1_ref, b1_ref, w2_ref, g_ref):
    dinv = dinv_ref[...]
    a1 = a1p_ref[0] + a1p_ref[1]
    out1 = a1 * dinv + x_ref[...] * (dinv * dinv)
    h = jnp.dot(out1, w1_ref[...])
    h = jnp.maximum(h + b1_ref[...], 0.0)
    p = jnp.dot(h, w2_ref[...])
    g_ref[...] = p * dinv


_tc1 = pl.pallas_call(
    _tc1_body,
    grid=(_NP // _R,),
    in_specs=[
        pl.BlockSpec((2, _R, _DIN), lambda i: (0, i, 0)),
        pl.BlockSpec((_R, _DIN), lambda i: (i, 0)),
        pl.BlockSpec((_R, 1), lambda i: (i, 0)),
        pl.BlockSpec((_DIN, _HID), lambda i: (0, 0)),
        pl.BlockSpec((_HID,), lambda i: (0,)),
        pl.BlockSpec((_HID, _WPAD), lambda i: (0, 0)),
    ],
    out_specs=pl.BlockSpec((_R, _WPAD), lambda i: (i, 0)),
    out_shape=jax.ShapeDtypeStruct((_NP, _WPAD), jnp.float32),
)


def _tc2_body(a2p_ref, g_ref, dinv_ref, b2_ref, o_ref):
    dinv = dinv_ref[...]
    s = (a2p_ref[0, :, :_WPAD] + a2p_ref[1, :, :_WPAD] + g_ref[...]) * dinv
    o_ref[...] = s[:, :_NCLS] + b2_ref[...]


_tc2 = pl.pallas_call(
    _tc2_body,
    grid=(_NP // _R,),
    in_specs=[
        pl.BlockSpec((2, _R, 128), lambda i: (0, i, 0)),
        pl.BlockSpec((_R, _WPAD), lambda i: (i, 0)),
        pl.BlockSpec((_R, 1), lambda i: (i, 0)),
        pl.BlockSpec((_NCLS,), lambda i: (0,)),
    ],
    out_specs=pl.BlockSpec((_R, _NCLS), lambda i: (i, 0)),
    out_shape=jax.ShapeDtypeStruct((_NP, _NCLS), jnp.float32),
)


def kernel(x, edge_index, edge_attr, W1, b1, W2, b2):
    pad = _EP - _E
    fill = jnp.arange(pad, dtype=jnp.int32)
    src = jnp.concatenate([edge_index[0], fill]).reshape(_NCHT, _B)
    dst = jnp.concatenate([edge_index[1], fill]).reshape(_NCHT, _B)
    ew = jnp.pad(edge_attr, (0, pad)).reshape(_NCHT, _B)
    w2p = jnp.pad(W2, ((0, 0), (0, _WPAD - _NCLS)))
    xp = jnp.pad(x, ((0, _NP - _N), (0, 0)))

    degp = _deg_kernel(dst, ew)
    dinv, y0, y1 = _tc0(degp, xp)
    a1p = _agg_l1(y0, y1, src, dst, ew)
    g = _tc1(a1p, xp, dinv, W1, b1, w2p)
    a2p = _agg_l2(g, src, dst, ew)
    return _tc2(a2p, g, dinv, b2)[:_N]


# speedup vs baseline: 2.5193x; 1.0366x over previous
import dataclasses
import functools

import jax
import jax.numpy as jnp
from jax import lax
from jax.experimental import pallas as pl
from jax.experimental.pallas import tpu as pltpu
from jax.experimental.pallas import tpu_sc as plsc

_N = 10000
_NP = 10240
_E = 320000
_DIN = 128
_HID = 256
_NCLS = 40
_WPAD = 48

_NC = 2
_NS = 16
_NW = _NC * _NS
_B = 128
_NCHT = 2560
_EP = _NCHT * _B
_NCH = _NCHT // _NW
_RPS = _NP // _NS

_mesh = plsc.VectorSubcoreMesh(core_axis_name="c", subcore_axis_name="s")

_sc_params = pltpu.CompilerParams(use_tc_tiling_on_sc=False)
if "needs_layout_passes" in pltpu.CompilerParams.__dataclass_fields__:
    _sc_params = dataclasses.replace(_sc_params, needs_layout_passes=False)


def _make_agg(width, npass):
    nsub = width // 16

    @functools.partial(
        pl.kernel,
        out_type=jax.ShapeDtypeStruct((_NC, _NP, 128), jnp.float32),
        mesh=_mesh,
        compiler_params=_sc_params,
        scratch_types=[
            pltpu.VMEM((_NCH, _B), jnp.int32),
            pltpu.VMEM((_NCH, _B), jnp.int32),
            pltpu.VMEM((_NCH, _B), jnp.float32),
            pltpu.VMEM((4, _B, width), jnp.float32),
            pltpu.VMEM_SHARED((_NP, width), jnp.float32),
        ] + [pltpu.SemaphoreType.DMA] * 8,
    )
    def agg(*refs):
        y_hbms = refs[:npass]
        src_hbm, dst_hbm, ew_hbm, out_hbm = refs[npass:npass + 4]
        src_v, dst_v, ew_v, rows_v, acc_s = refs[npass + 4:npass + 9]
        gsem = refs[npass + 9:npass + 13]
        ssem = refs[npass + 13:npass + 17]
        cid = lax.axis_index("c")
        sid = lax.axis_index("s")
        wid = cid * _NS + sid

        pltpu.sync_copy(src_hbm.at[pl.ds(wid * _NCH, _NCH)], src_v)
        pltpu.sync_copy(dst_hbm.at[pl.ds(wid * _NCH, _NCH)], dst_v)
        pltpu.sync_copy(ew_hbm.at[pl.ds(wid * _NCH, _NCH)], ew_v)

        zero = jnp.zeros((16,), jnp.float32)

        for h in range(npass):
            @pl.loop(0, _B)
            def _(r):
                for k in range(nsub):
                    rows_v.at[0][r, pl.ds(k * 16, 16)] = zero

            for z in range(_RPS // _B):
                pltpu.sync_copy(rows_v.at[0],
                                acc_s.at[pl.ds(sid * _RPS + z * _B, _B)])
            plsc.subcore_barrier()

            y_hbm = y_hbms[h]

            def start_gather(c, j):
                pltpu.async_copy(y_hbm.at[src_v.at[c]], rows_v.at[j], gsem[j])

            def wait_gather(j):
                pltpu.make_async_copy(
                    y_hbm.at[src_v.at[0]], rows_v.at[j], gsem[j]).wait()

            def start_scatter(c, j):
                pltpu.async_copy(rows_v.at[j], acc_s.at[dst_v.at[c]],
                                 ssem[j], add=True)

            def wait_scatter(j):
                pltpu.make_async_copy(
                    rows_v.at[j], acc_s.at[dst_v.at[0]], ssem[j]).wait()

            def compute(c, j):
                cvec = lax.broadcast_in_dim(c, (16,), ())

                @plsc.parallel_loop(0, _B, unroll=8)
                def _(e):
                    w = plsc.load_gather(
                        ew_v, [cvec, lax.broadcast_in_dim(e, (16,), ())])
                    for k in range(nsub):
                        sl = (e, pl.ds(k * 16, 16))
                        rows_v.at[j][sl] = rows_v.at[j][sl] * w

            start_gather(0, 0)
            start_gather(1, 1)
            wait_gather(0)
            compute(0, 0)
            start_scatter(0, 0)
            start_gather(2, 2)
            wait_gather(1)
            compute(1, 1)
            start_scatter(1, 1)
            start_gather(3, 3)

            @pl.loop(0, 19)
            def _(m):
                for j in range(4):
                    c = 4 * m + 2 + j
                    b = (j + 2) % 4
                    bn = j
                    wait_gather(b)
                    compute(c, b)
                    start_scatter(c, b)
                    wait_scatter(bn)
                    start_gather(c + 2, bn)

            wait_gather(2)
            compute(_NCH - 2, 2)
            start_scatter(_NCH - 2, 2)
            wait_scatter(0)
            wait_gather(3)
            compute(_NCH - 1, 3)
            start_scatter(_NCH - 1, 3)
            wait_scatter(1)
            wait_scatter(2)
            wait_scatter(3)

            plsc.subcore_barrier()
            pltpu.sync_copy(
                acc_s.at[pl.ds(sid * _RPS, _RPS)],
                out_hbm.at[cid, pl.ds(sid * _RPS, _RPS),
                           pl.ds(h * width, width)])

    return agg


_HW = _DIN // 2
_agg_l1 = _make_agg(_HW, 2)
_agg_l2 = _make_agg(_WPAD, 1)


@functools.partial(
    pl.kernel,
    out_type=jax.ShapeDtypeStruct((_NW, _NP // 128, 128), jnp.float32),
    mesh=_mesh,
    compiler_params=_sc_params,
    scratch_types=[
        pltpu.VMEM((_NCH, _B), jnp.int32),
        pltpu.VMEM((_NCH, _B), jnp.float32),
        pltpu.VMEM((_NP // 128, 128), jnp.float32),
    ],
)
def _deg_kernel(dst_hbm, ew_hbm, out_hbm, dst_v, ew_v, acc_v):
    cid = lax.axis_index("c")
    sid = lax.axis_index("s")
    wid = cid * _NS + sid

    pltpu.sync_copy(dst_hbm.at[pl.ds(wid * _NCH, _NCH)], dst_v)
    pltpu.sync_copy(ew_hbm.at[pl.ds(wid * _NCH, _NCH)], ew_v)

    zero = jnp.zeros((16,), jnp.float32)

    @pl.loop(0, _NP // 128)
    def _(r):
        for k in range(8):
            acc_v[r, pl.ds(k * 16, 16)] = zero

    @pl.loop(0, _NCH)
    def _(c):
        @pl.loop(0, _B, step=16)
        def _(e):
            d = dst_v[c, pl.ds(e, 16)]
            w = ew_v[c, pl.ds(e, 16)]
            plsc.addupdate_scatter(
                acc_v, [lax.shift_right_logical(d, 7),
                        lax.bitwise_and(d, 127)], w)

    pltpu.sync_copy(acc_v, out_hbm.at[wid])


_R = 1024


def _tc0_body(degp_ref, x_ref, dinv_ref, y0_ref, y1_ref):
    dinv = lax.rsqrt(degp_ref[...] + 1.0)
    dinv_ref[...] = dinv
    y = x_ref[...] * dinv
    y0_ref[...] = y[:, :_HW]
    y1_ref[...] = y[:, _HW:]


_tc0 = pl.pallas_call(
    _tc0_body,
    grid=(_NP // _R,),
    in_specs=[
        pl.BlockSpec((_R, 1), lambda i: (i, 0)),
        pl.BlockSpec((_R, _DIN), lambda i: (i, 0)),
    ],
    out_specs=[
        pl.BlockSpec((_R, 1), lambda i: (i, 0)),
        pl.BlockSpec((_R, _HW), lambda i: (i, 0)),
        pl.BlockSpec((_R, _HW), lambda i: (i, 0)),
    ],
    out_shape=[
        jax.ShapeDtypeStruct((_NP, 1), jnp.float32),
        jax.ShapeDtypeStruct((_NP, _HW), jnp.float32),
        jax.ShapeDtypeStruct((_NP, _HW), jnp.float32),
    ],
)


def _tc1_body(a1p_ref, x_ref, dinv_ref, w1_ref, b1_ref, w2_ref, g_ref):
    dinv = dinv_ref[...]
    a1 = a1p_ref[0] + a1p_ref[1]
    out1 = a1 * dinv + x_ref[...] * (dinv * dinv)
    h = jnp.dot(out1, w1_ref[...])
    h = jnp.maximum(h + b1_ref[...], 0.0)
    p = jnp.dot(h, w2_ref[...])
    g_ref[...] = p * dinv


_tc1 = pl.pallas_call(
    _tc1_body,
    grid=(_NP // _R,),
    in_specs=[
        pl.BlockSpec((2, _R, _DIN), lambda i: (0, i, 0)),
        pl.BlockSpec((_R, _DIN), lambda i: (i, 0)),
        pl.BlockSpec((_R, 1), lambda i: (i, 0)),
        pl.BlockSpec((_DIN, _HID), lambda i: (0, 0)),
        pl.BlockSpec((_HID,), lambda i: (0,)),
        pl.BlockSpec((_HID, _WPAD), lambda i: (0, 0)),
    ],
    out_specs=pl.BlockSpec((_R, _WPAD), lambda i: (i, 0)),
    out_shape=jax.ShapeDtypeStruct((_NP, _WPAD), jnp.float32),
)


def _tc2_body(a2p_ref, g_ref, dinv_ref, b2_ref, o_ref):
    dinv = dinv_ref[...]
    s = (a2p_ref[0, :, :_WPAD] + a2p_ref[1, :, :_WPAD] + g_ref[...]) * dinv
    o_ref[...] = s[:, :_NCLS] + b2_ref[...]


_tc2 = pl.pallas_call(
    _tc2_body,
    grid=(_NP // _R,),
    in_specs=[
        pl.BlockSpec((2, _R, 128), lambda i: (0, i, 0)),
        pl.BlockSpec((_R, _WPAD), lambda i: (i, 0)),
        pl.BlockSpec((_R, 1), lambda i: (i, 0)),
        pl.BlockSpec((_NCLS,), lambda i: (0,)),
    ],
    out_specs=pl.BlockSpec((_R, _NCLS), lambda i: (i, 0)),
    out_shape=jax.ShapeDtypeStruct((_NP, _NCLS), jnp.float32),
)


def kernel(x, edge_index, edge_attr, W1, b1, W2, b2):
    pad = _EP - _E
    fill = jnp.arange(pad, dtype=jnp.int32)
    src = jnp.concatenate([edge_index[0], fill]).reshape(_NCHT, _B)
    dst = jnp.concatenate([edge_index[1], fill]).reshape(_NCHT, _B)
    ew = jnp.pad(edge_attr, (0, pad)).reshape(_NCHT, _B)
    w2p = jnp.pad(W2, ((0, 0), (0, _WPAD - _NCLS)))
    xp = jnp.pad(x, ((0, _NP - _N), (0, 0)))

    degp = _deg_kernel(dst, ew)
    degsum = jnp.sum(degp, axis=0).reshape(_NP, 1)
    dinv, y0, y1 = _tc0(degsum, xp)
    a1p = _agg_l1(y0, y1, src, dst, ew)
    g = _tc1(a1p, xp, dinv, W1, b1, w2p)
    a2p = _agg_l2(g, src, dst, ew)
    return _tc2(a2p, g, dinv, b2)[:_N]


# speedup vs baseline: 2.5562x; 1.0146x over previous
import dataclasses
import functools

import jax
import jax.numpy as jnp
from jax import lax
from jax.experimental import pallas as pl
from jax.experimental.pallas import tpu as pltpu
from jax.experimental.pallas import tpu_sc as plsc

_N = 10000
_NP = 10240
_E = 320000
_DIN = 128
_HID = 256
_NCLS = 40
_WPAD = 48

_NC = 2
_NS = 16
_NW = _NC * _NS
_B = 128
_NCHT = 2560
_EP = _NCHT * _B
_NCH = _NCHT // _NW
_RPS = _NP // _NS

_mesh = plsc.VectorSubcoreMesh(core_axis_name="c", subcore_axis_name="s")

_sc_params = pltpu.CompilerParams(use_tc_tiling_on_sc=False)
if "needs_layout_passes" in pltpu.CompilerParams.__dataclass_fields__:
    _sc_params = dataclasses.replace(_sc_params, needs_layout_passes=False)


def _make_agg(width, npass, bf16_src=False):
    nsub = width // 16
    gdtype = jnp.bfloat16 if bf16_src else jnp.float32

    @functools.partial(
        pl.kernel,
        out_type=jax.ShapeDtypeStruct((_NC, _NP, 128), jnp.float32),
        mesh=_mesh,
        compiler_params=_sc_params,
        scratch_types=[
            pltpu.VMEM((_NCH, _B), jnp.int32),
            pltpu.VMEM((_NCH, _B), jnp.int32),
            pltpu.VMEM((_NCH, _B), jnp.float32),
            pltpu.VMEM((4, _B, width), gdtype),
            pltpu.VMEM((4, _B, width), jnp.float32),
            pltpu.VMEM_SHARED((_NP, width), jnp.float32),
        ] + [pltpu.SemaphoreType.DMA] * 8,
    )
    def agg(*refs):
        y_hbms = refs[:npass]
        src_hbm, dst_hbm, ew_hbm, out_hbm = refs[npass:npass + 4]
        src_v, dst_v, ew_v, grows_v, rows_v, acc_s = refs[npass + 4:npass + 10]
        gsem = refs[npass + 10:npass + 14]
        ssem = refs[npass + 14:npass + 18]
        cid = lax.axis_index("c")
        sid = lax.axis_index("s")
        wid = cid * _NS + sid

        pltpu.sync_copy(src_hbm.at[pl.ds(wid * _NCH, _NCH)], src_v)
        pltpu.sync_copy(dst_hbm.at[pl.ds(wid * _NCH, _NCH)], dst_v)
        pltpu.sync_copy(ew_hbm.at[pl.ds(wid * _NCH, _NCH)], ew_v)

        zero = jnp.zeros((16,), jnp.float32)

        for h in range(npass):
            @pl.loop(0, _B)
            def _(r):
                for k in range(nsub):
                    rows_v.at[0][r, pl.ds(k * 16, 16)] = zero

            for z in range(_RPS // _B):
                pltpu.sync_copy(rows_v.at[0],
                                acc_s.at[pl.ds(sid * _RPS + z * _B, _B)])
            plsc.subcore_barrier()

            y_hbm = y_hbms[h]

            def start_gather(c, j):
                pltpu.async_copy(y_hbm.at[src_v.at[c]], grows_v.at[j], gsem[j])

            def wait_gather(j):
                pltpu.make_async_copy(
                    y_hbm.at[src_v.at[0]], grows_v.at[j], gsem[j]).wait()

            def start_scatter(c, j):
                pltpu.async_copy(rows_v.at[j], acc_s.at[dst_v.at[c]],
                                 ssem[j], add=True)

            def wait_scatter(j):
                pltpu.make_async_copy(
                    rows_v.at[j], acc_s.at[dst_v.at[0]], ssem[j]).wait()

            if bf16_src:
                def compute(c, j):
                    cvec = lax.broadcast_in_dim(c, (16,), ())

                    @plsc.parallel_loop(0, _B, unroll=8)
                    def _(e):
                        w = plsc.load_gather(
                            ew_v, [cvec, lax.broadcast_in_dim(e, (16,), ())])
                        for k in range(nsub // 2):
                            v = grows_v.at[j][e, pl.ds(k * 32, 32)]
                            a, b = plsc.unpack(
                                v, format=plsc.PackFormat.INTERLEAVED,
                                preferred_element_type=jnp.float32)
                            rows_v.at[j][e, pl.ds(k * 32, 16)] = a * w
                            rows_v.at[j][e, pl.ds(k * 32 + 16, 16)] = b * w
            else:
                def compute(c, j):
                    cvec = lax.broadcast_in_dim(c, (16,), ())

                    @plsc.parallel_loop(0, _B, unroll=8)
                    def _(e):
                        w = plsc.load_gather(
                            ew_v, [cvec, lax.broadcast_in_dim(e, (16,), ())])
                        for k in range(nsub):
                            sl = (e, pl.ds(k * 16, 16))
                            v = grows_v.at[j][sl]
                            rows_v.at[j][sl] = v * w

            start_gather(0, 0)
            start_gather(1, 1)
            wait_gather(0)
            compute(0, 0)
            start_scatter(0, 0)
            start_gather(2, 2)
            wait_gather(1)
            compute(1, 1)
            start_scatter(1, 1)
            start_gather(3, 3)

            @pl.loop(0, 19)
            def _(m):
                for j in range(4):
                    c = 4 * m + 2 + j
                    b = (j + 2) % 4
                    bn = j
                    wait_gather(b)
                    compute(c, b)
                    start_scatter(c, b)
                    wait_scatter(bn)
                    start_gather(c + 2, bn)

            wait_gather(2)
            compute(_NCH - 2, 2)
            start_scatter(_NCH - 2, 2)
            wait_scatter(0)
            wait_gather(3)
            compute(_NCH - 1, 3)
            start_scatter(_NCH - 1, 3)
            wait_scatter(1)
            wait_scatter(2)
            wait_scatter(3)

            plsc.subcore_barrier()
            pltpu.sync_copy(
                acc_s.at[pl.ds(sid * _RPS, _RPS)],
                out_hbm.at[cid, pl.ds(sid * _RPS, _RPS),
                           pl.ds(h * width, width)])

    return agg


_HW = _DIN // 2
_agg_l1 = _make_agg(_HW, 2, bf16_src=True)
_agg_l2 = _make_agg(_WPAD, 1)

_ILV = sum([[32 * k + i, 32 * k + 16 + i] for k in range(_HW // 32)
            for i in range(16)], [])


@functools.partial(
    pl.kernel,
    out_type=jax.ShapeDtypeStruct((_NW, _NP // 128, 128), jnp.float32),
    mesh=_mesh,
    compiler_params=_sc_params,
    scratch_types=[
        pltpu.VMEM((_NCH, _B), jnp.int32),
        pltpu.VMEM((_NCH, _B), jnp.float32),
        pltpu.VMEM((_NP // 128, 128), jnp.float32),
    ],
)
def _deg_kernel(dst_hbm, ew_hbm, out_hbm, dst_v, ew_v, acc_v):
    cid = lax.axis_index("c")
    sid = lax.axis_index("s")
    wid = cid * _NS + sid

    pltpu.sync_copy(dst_hbm.at[pl.ds(wid * _NCH, _NCH)], dst_v)
    pltpu.sync_copy(ew_hbm.at[pl.ds(wid * _NCH, _NCH)], ew_v)

    zero = jnp.zeros((16,), jnp.float32)

    @pl.loop(0, _NP // 128)
    def _(r):
        for k in range(8):
            acc_v[r, pl.ds(k * 16, 16)] = zero

    @pl.loop(0, _NCH)
    def _(c):
        @pl.loop(0, _B, step=16)
        def _(e):
            d = dst_v[c, pl.ds(e, 16)]
            w = ew_v[c, pl.ds(e, 16)]
            plsc.addupdate_scatter(
                acc_v, [lax.shift_right_logical(d, 7),
                        lax.bitwise_and(d, 127)], w)

    pltpu.sync_copy(acc_v, out_hbm.at[wid])


_R = 1024


def _tc0_body(degp_ref, x_ref, dinv_ref, y0_ref, y1_ref):
    dinv = lax.rsqrt(degp_ref[...] + 1.0)
    dinv_ref[...] = dinv
    y = x_ref[...] * dinv
    y0_ref[...] = y[:, :_HW]
    y1_ref[...] = y[:, _HW:]


_tc0 = pl.pallas_call(
    _tc0_body,
    grid=(_NP // _R,),
    in_specs=[
        pl.BlockSpec((_R, 1), lambda i: (i, 0)),
        pl.BlockSpec((_R, _DIN), lambda i: (i, 0)),
    ],
    out_specs=[
        pl.BlockSpec((_R, 1), lambda i: (i, 0)),
        pl.BlockSpec((_R, _HW), lambda i: (i, 0)),
        pl.BlockSpec((_R, _HW), lambda i: (i, 0)),
    ],
    out_shape=[
        jax.ShapeDtypeStruct((_NP, 1), jnp.float32),
        jax.ShapeDtypeStruct((_NP, _HW), jnp.float32),
        jax.ShapeDtypeStruct((_NP, _HW), jnp.float32),
    ],
)


def _tc1_body(a1p_ref, x_ref, dinv_ref, w1_ref, b1_ref, w2_ref, g_ref):
    dinv = dinv_ref[...]
    a1 = a1p_ref[0] + a1p_ref[1]
    out1 = a1 * dinv + x_ref[...] * (dinv * dinv)
    h = jnp.dot(out1, w1_ref[...])
    h = jnp.maximum(h + b1_ref[...], 0.0)
    p = jnp.dot(h, w2_ref[...])
    g_ref[...] = p * dinv


_tc1 = pl.pallas_call(
    _tc1_body,
    grid=(_NP // _R,),
    in_specs=[
        pl.BlockSpec((2, _R, _DIN), lambda i: (0, i, 0)),
        pl.BlockSpec((_R, _DIN), lambda i: (i, 0)),
        pl.BlockSpec((_R, 1), lambda i: (i, 0)),
        pl.BlockSpec((_DIN, _HID), lambda i: (0, 0)),
        pl.BlockSpec((_HID,), lambda i: (0,)),
        pl.BlockSpec((_HID, _WPAD), lambda i: (0, 0)),
    ],
    out_specs=pl.BlockSpec((_R, _WPAD), lambda i: (i, 0)),
    out_shape=jax.ShapeDtypeStruct((_NP, _WPAD), jnp.float32),
)


def _tc2_body(a2p_ref, g_ref, dinv_ref, b2_ref, o_ref):
    dinv = dinv_ref[...]
    s = (a2p_ref[0, :, :_WPAD] + a2p_ref[1, :, :_WPAD] + g_ref[...]) * dinv
    o_ref[...] = s[:, :_NCLS] + b2_ref[...]


_tc2 = pl.pallas_call(
    _tc2_body,
    grid=(_NP // _R,),
    in_specs=[
        pl.BlockSpec((2, _R, 128), lambda i: (0, i, 0)),
        pl.BlockSpec((_R, _WPAD), lambda i: (i, 0)),
        pl.BlockSpec((_R, 1), lambda i: (i, 0)),
        pl.BlockSpec((_NCLS,), lambda i: (0,)),
    ],
    out_specs=pl.BlockSpec((_R, _NCLS), lambda i: (i, 0)),
    out_shape=jax.ShapeDtypeStruct((_NP, _NCLS), jnp.float32),
)


def kernel(x, edge_index, edge_attr, W1, b1, W2, b2):
    pad = _EP - _E
    fill = jnp.arange(pad, dtype=jnp.int32)
    src = jnp.concatenate([edge_index[0], fill]).reshape(_NCHT, _B)
    dst = jnp.concatenate([edge_index[1], fill]).reshape(_NCHT, _B)
    ew = jnp.pad(edge_attr, (0, pad)).reshape(_NCHT, _B)
    w2p = jnp.pad(W2, ((0, 0), (0, _WPAD - _NCLS)))
    xp = jnp.pad(x, ((0, _NP - _N), (0, 0)))

    degp = _deg_kernel(dst, ew)
    degsum = jnp.sum(degp, axis=0).reshape(_NP, 1)
    dinv, y0, y1 = _tc0(degsum, xp)
    ilv = jnp.asarray(_ILV, jnp.int32)
    y0b = jnp.take(y0, ilv, axis=1).astype(jnp.bfloat16)
    y1b = jnp.take(y1, ilv, axis=1).astype(jnp.bfloat16)
    a1p = _agg_l1(y0b, y1b, src, dst, ew)
    g = _tc1(a1p, xp, dinv, W1, b1, w2p)
    a2p = _agg_l2(g, src, dst, ew)
    return _tc2(a2p, g, dinv, b2)[:_N]
